# baseline (speedup 1.0000x reference)
import functools

import jax
import jax.numpy as jnp
from jax import lax
from jax.experimental import pallas as pl
from jax.experimental.pallas import tpu as pltpu
from jax.experimental.pallas import tpu_sc as plsc

_B, _T, _D = 8, 4, 2048
_E, _TOPK, _I, _NS = 8, 2, 512, 2
_N = _B * _T
_SLOTS = _NS + _E
_L = 16
_KC = _D // _L


def _sweep_body(x_ref, gwT_ref, w1s_ref, w3s_ref, w2s_ref,
                w1r_ref, w3r_ref, w2r_ref, logits_ref, shared_ref, cexp_ref):
    s = pl.program_id(0)
    xv = x_ref[...]

    @pl.when(s < _NS)
    def _shared():
        h = jax.nn.silu(jnp.dot(xv, w1s_ref[...], preferred_element_type=jnp.float32)
                        * jnp.dot(xv, w3s_ref[...], preferred_element_type=jnp.float32))
        contrib = jnp.dot(h, w2s_ref[...], preferred_element_type=jnp.float32)

        @pl.when(s == 0)
        def _init():
            logits_ref[...] = jnp.dot(xv, gwT_ref[...],
                                      preferred_element_type=jnp.float32)
            shared_ref[...] = contrib

        @pl.when(s > 0)
        def _acc():
            shared_ref[...] += contrib

    @pl.when(s >= _NS)
    def _routed():
        h = jax.nn.silu(jnp.dot(xv, w1r_ref[0], preferred_element_type=jnp.float32)
                        * jnp.dot(xv, w3r_ref[0], preferred_element_type=jnp.float32))
        cexp_ref[0] = jnp.dot(h, w2r_ref[0], preferred_element_type=jnp.float32)


def _routing_body(logits_hbm, gb_hbm, shared_hbm, cexp_hbm,
                  scores_hbm, out_hbm, l_v, gb_v, sc_v, sh_v, ce_v, out_v):
    t = lax.axis_index("s") * 2 + lax.axis_index("c")
    pltpu.sync_copy(logits_hbm.at[t], l_v)
    pltpu.sync_copy(gb_hbm, gb_v)
    pltpu.sync_copy(shared_hbm.at[t], sh_v)
    for e in range(_E):
        pltpu.sync_copy(cexp_hbm.at[e, t], ce_v.at[e])

    iota = lax.iota(jnp.int32, _L)
    logits = l_v[...]
    sig = 1.0 / (1.0 + jnp.exp(-logits))
    scores = sig + gb_v[...]
    sc_v[...] = scores

    ninf = jnp.float32(-jnp.inf)
    z = jnp.where(iota < _E, scores, ninf)
    m1 = jnp.max(z)
    t1i = jnp.min(jnp.where(z == m1, iota, _L))
    masked = jnp.where(iota == t1i, ninf, z)
    m2 = jnp.max(masked)
    t2i = jnp.min(jnp.where(masked == m2, iota, _L))

    pltpu.sync_copy(sc_v, scores_hbm.at[t])

    zero = jnp.zeros((_L,), jnp.float32)
    ge = [jnp.where(t1i == e, zero + m1, zero)
          + jnp.where(t2i == e, zero + m2, zero) for e in range(_E)]
    denom = m1 + m2
    for k in range(_KC):
        sl = pl.ds(k * _L, _L)
        acc = ge[0] * ce_v[0, sl]
        for e in range(1, _E):
            acc = acc + ge[e] * ce_v[e, sl]
        out_v[sl] = sh_v[sl] + acc / denom
    pltpu.sync_copy(out_v, out_hbm.at[t])


def _routing_combine(logits16, gb_pad, shared2d, cexp):
    mesh = plsc.VectorSubcoreMesh(core_axis_name="c", subcore_axis_name="s")
    return pl.kernel(
        _routing_body,
        mesh=mesh,
        out_type=[jax.ShapeDtypeStruct((_N, _L), jnp.float32),
                  jax.ShapeDtypeStruct((_N, _D), jnp.float32)],
        scratch_types=[pltpu.VMEM((_L,), jnp.float32),
                       pltpu.VMEM((_L,), jnp.float32),
                       pltpu.VMEM((_L,), jnp.float32),
                       pltpu.VMEM((_D,), jnp.float32),
                       pltpu.VMEM((_E, _D), jnp.float32),
                       pltpu.VMEM((_D,), jnp.float32)],
        compiler_params=pltpu.CompilerParams(needs_layout_passes=False),
    )(logits16, gb_pad, shared2d, cexp)


@functools.partial(jax.jit, static_argnames=())
def kernel(x, w1_shared, w2_shared, w3_shared, w1_routed, w2_routed, w3_routed,
           gate_w, gate_b):
    x2d = x.reshape(_N, _D)
    gwT_pad = jnp.pad(gate_w.T, ((0, 0), (0, _L - _E)))
    gb_pad = jnp.pad(gate_b, (0, _L - _E))

    def _c(i):
        return jnp.clip(i - _NS, 0, _E - 1)

    logits16, shared2d, cexp = pl.pallas_call(
        _sweep_body,
        grid=(_SLOTS,),
        in_specs=[
            pl.BlockSpec((_N, _D), lambda i: (0, 0)),
            pl.BlockSpec((_D, _L), lambda i: (0, 0)),
            pl.BlockSpec((_D, _I), lambda i: (0, jnp.clip(i, 0, _NS - 1))),
            pl.BlockSpec((_D, _I), lambda i: (0, jnp.clip(i, 0, _NS - 1))),
            pl.BlockSpec((_I, _D), lambda i: (jnp.clip(i, 0, _NS - 1), 0)),
            pl.BlockSpec((1, _D, _I), lambda i: (_c(i), 0, 0)),
            pl.BlockSpec((1, _D, _I), lambda i: (_c(i), 0, 0)),
            pl.BlockSpec((1, _I, _D), lambda i: (_c(i), 0, 0)),
        ],
        out_specs=[
            pl.BlockSpec((_N, _L), lambda i: (0, 0)),
            pl.BlockSpec((_N, _D), lambda i: (0, 0)),
            pl.BlockSpec((1, _N, _D), lambda i: (_c(i), 0, 0)),
        ],
        out_shape=[
            jax.ShapeDtypeStruct((_N, _L), jnp.float32),
            jax.ShapeDtypeStruct((_N, _D), jnp.float32),
            jax.ShapeDtypeStruct((_E, _N, _D), jnp.float32),
        ],
        compiler_params=pltpu.CompilerParams(
            dimension_semantics=("arbitrary",),
        ),
    )(x2d, gwT_pad, w1_shared, w3_shared, w2_shared,
      w1_routed, w3_routed, w2_routed)

    scores16, out2d = _routing_combine(logits16, gb_pad, shared2d, cexp)

    return (out2d.reshape(_B, _T, _D), scores16[:, :_E].reshape(_B, _T, _E))


# speedup vs baseline: 1.1736x; 1.1736x over previous
import functools

import jax
import jax.numpy as jnp
from jax import lax
from jax.experimental import pallas as pl
from jax.experimental.pallas import tpu as pltpu
from jax.experimental.pallas import tpu_sc as plsc

_B, _T, _D = 8, 4, 2048
_E, _TOPK, _I, _NS = 8, 2, 512, 2
_N = _B * _T
_L = 16


def _gate_body(x_ref, gwT_ref, logits_ref):
    logits_ref[...] = jnp.dot(x_ref[...], gwT_ref[...],
                              preferred_element_type=jnp.float32)


def _routing_body(logits_hbm, gb_hbm, scores_hbm, g_hbm, l_v, gb_v, sc_v, g_v):
    t = lax.axis_index("s") * 2 + lax.axis_index("c")
    pltpu.sync_copy(logits_hbm.at[t], l_v)
    pltpu.sync_copy(gb_hbm, gb_v)

    iota = lax.iota(jnp.int32, _L)
    logits = l_v[...]
    sig = 1.0 / (1.0 + jnp.exp(-logits))
    scores = sig + gb_v[...]
    sc_v[...] = scores

    ninf = jnp.float32(-jnp.inf)
    z = jnp.where(iota < _E, scores, ninf)
    m1 = jnp.max(z)
    t1i = jnp.min(jnp.where(z == m1, iota, _L))
    masked = jnp.where(iota == t1i, ninf, z)
    m2 = jnp.max(masked)
    t2i = jnp.min(jnp.where(masked == m2, iota, _L))
    num = jnp.where(iota == t1i, m1, 0.0) + jnp.where(iota == t2i, m2, 0.0)
    g_v[...] = num / (m1 + m2)

    pltpu.sync_copy(sc_v, scores_hbm.at[t])
    pltpu.sync_copy(g_v, g_hbm.at[t])


def _routing(logits16, gb_pad):
    mesh = plsc.VectorSubcoreMesh(core_axis_name="c", subcore_axis_name="s")
    return pl.kernel(
        _routing_body,
        mesh=mesh,
        out_type=[jax.ShapeDtypeStruct((_N, _L), jnp.float32),
                  jax.ShapeDtypeStruct((_N, _L), jnp.float32)],
        scratch_types=[pltpu.VMEM((_L,), jnp.float32),
                       pltpu.VMEM((_L,), jnp.float32),
                       pltpu.VMEM((_L,), jnp.float32),
                       pltpu.VMEM((_L,), jnp.float32)],
        compiler_params=pltpu.CompilerParams(needs_layout_passes=False),
    )(logits16, gb_pad)


def _shared_body(x_ref, w1s_ref, w3s_ref, w2s_ref, out_ref):
    s = pl.program_id(0)
    xv = x_ref[...]
    h = jax.nn.silu(jnp.dot(xv, w1s_ref[...], preferred_element_type=jnp.float32)
                    * jnp.dot(xv, w3s_ref[...], preferred_element_type=jnp.float32))
    contrib = jnp.dot(h, w2s_ref[...], preferred_element_type=jnp.float32)

    @pl.when(s == 0)
    def _init():
        out_ref[...] = contrib

    @pl.when(s > 0)
    def _acc():
        out_ref[...] += contrib


def _routed_body(x_ref, g_ref, prev_ref, w1r_ref, w3r_ref, w2r_ref, out_ref):
    s = pl.program_id(0)
    xv = x_ref[...]
    iota = jax.lax.broadcasted_iota(jnp.int32, (_N, _L), 1)
    wtok = jnp.sum(jnp.where(iota == s, g_ref[...], 0.0),
                   axis=1, keepdims=True)
    h = jax.nn.silu(jnp.dot(xv, w1r_ref[0], preferred_element_type=jnp.float32)
                    * jnp.dot(xv, w3r_ref[0], preferred_element_type=jnp.float32))
    contrib = jnp.dot(wtok * h, w2r_ref[0], preferred_element_type=jnp.float32)

    @pl.when(s == 0)
    def _init():
        out_ref[...] = prev_ref[...] + contrib

    @pl.when(s > 0)
    def _acc():
        out_ref[...] += contrib


@functools.partial(jax.jit, static_argnames=())
def kernel(x, w1_shared, w2_shared, w3_shared, w1_routed, w2_routed, w3_routed,
           gate_w, gate_b):
    x2d = x.reshape(_N, _D)
    gwT_pad = jnp.pad(gate_w.T, ((0, 0), (0, _L - _E)))
    gb_pad = jnp.pad(gate_b, (0, _L - _E))

    logits16 = pl.pallas_call(
        _gate_body,
        out_shape=jax.ShapeDtypeStruct((_N, _L), jnp.float32),
    )(x2d, gwT_pad)

    scores16, g16 = _routing(logits16, gb_pad)

    shared2d = pl.pallas_call(
        _shared_body,
        grid=(_NS,),
        in_specs=[
            pl.BlockSpec((_N, _D), lambda i: (0, 0)),
            pl.BlockSpec((_D, _I), lambda i: (0, i)),
            pl.BlockSpec((_D, _I), lambda i: (0, i)),
            pl.BlockSpec((_I, _D), lambda i: (i, 0)),
        ],
        out_specs=pl.BlockSpec((_N, _D), lambda i: (0, 0)),
        out_shape=jax.ShapeDtypeStruct((_N, _D), jnp.float32),
        compiler_params=pltpu.CompilerParams(
            dimension_semantics=("arbitrary",),
        ),
    )(x2d, w1_shared, w3_shared, w2_shared)

    out2d = pl.pallas_call(
        _routed_body,
        grid=(_E,),
        in_specs=[
            pl.BlockSpec((_N, _D), lambda i: (0, 0)),
            pl.BlockSpec((_N, _L), lambda i: (0, 0)),
            pl.BlockSpec((_N, _D), lambda i: (0, 0)),
            pl.BlockSpec((1, _D, _I), lambda i: (i, 0, 0)),
            pl.BlockSpec((1, _D, _I), lambda i: (i, 0, 0)),
            pl.BlockSpec((1, _I, _D), lambda i: (i, 0, 0)),
        ],
        out_specs=pl.BlockSpec((_N, _D), lambda i: (0, 0)),
        out_shape=jax.ShapeDtypeStruct((_N, _D), jnp.float32),
        compiler_params=pltpu.CompilerParams(
            dimension_semantics=("arbitrary",),
        ),
    )(x2d, g16, shared2d, w1_routed, w3_routed, w2_routed)

    return (out2d.reshape(_B, _T, _D), scores16[:, :_E].reshape(_B, _T, _E))


# speedup vs baseline: 1.1828x; 1.0078x over previous
import functools

import jax
import jax.numpy as jnp
from jax import lax
from jax.experimental import pallas as pl
from jax.experimental.pallas import tpu as pltpu
from jax.experimental.pallas import tpu_sc as plsc

_B, _T, _D = 8, 4, 2048
_E, _TOPK, _I, _NS = 8, 2, 512, 2
_N = _B * _T
_L = 16


def _gate_body(x_ref, gwT_ref, logits_ref):
    logits_ref[...] = jnp.dot(x_ref[...], gwT_ref[...],
                              preferred_element_type=jnp.float32)


def _routing_body(logits_hbm, gb_hbm, scores_hbm, g_hbm, l_v, gb_v, sc_v, g_v):
    t = lax.axis_index("s") * 2 + lax.axis_index("c")
    pltpu.sync_copy(logits_hbm.at[t], l_v)
    pltpu.sync_copy(gb_hbm, gb_v)

    iota = lax.iota(jnp.int32, _L)
    logits = l_v[...]
    sig = 1.0 / (1.0 + jnp.exp(-logits))
    scores = sig + gb_v[...]
    sc_v[...] = scores

    ninf = jnp.float32(-jnp.inf)
    z = jnp.where(iota < _E, scores, ninf)
    m1 = jnp.max(z)
    t1i = jnp.min(jnp.where(z == m1, iota, _L))
    masked = jnp.where(iota == t1i, ninf, z)
    m2 = jnp.max(masked)
    t2i = jnp.min(jnp.where(masked == m2, iota, _L))
    num = jnp.where(iota == t1i, m1, 0.0) + jnp.where(iota == t2i, m2, 0.0)
    g_v[...] = num / (m1 + m2)

    pltpu.sync_copy(sc_v, scores_hbm.at[t])
    pltpu.sync_copy(g_v, g_hbm.at[t])


def _routing(logits16, gb_pad):
    mesh = plsc.VectorSubcoreMesh(core_axis_name="c", subcore_axis_name="s")
    return pl.kernel(
        _routing_body,
        mesh=mesh,
        out_type=[jax.ShapeDtypeStruct((_N, _L), jnp.float32),
                  jax.ShapeDtypeStruct((_N, _L), jnp.float32)],
        scratch_types=[pltpu.VMEM((_L,), jnp.float32),
                       pltpu.VMEM((_L,), jnp.float32),
                       pltpu.VMEM((_L,), jnp.float32),
                       pltpu.VMEM((_L,), jnp.float32)],
        compiler_params=pltpu.CompilerParams(needs_layout_passes=False),
    )(logits16, gb_pad)


def _shared_body(x_ref, w1s_ref, w3s_ref, w2s_ref, out_ref):
    s = pl.program_id(0)
    xv = x_ref[...]
    h = jax.nn.silu(jnp.dot(xv, w1s_ref[...], preferred_element_type=jnp.float32)
                    * jnp.dot(xv, w3s_ref[...], preferred_element_type=jnp.float32))
    contrib = jnp.dot(h, w2s_ref[...], preferred_element_type=jnp.float32)

    @pl.when(s == 0)
    def _init():
        out_ref[...] = contrib

    @pl.when(s > 0)
    def _acc():
        out_ref[...] += contrib


def _routed_body(x_ref, g_ref, prev_ref, w1r_ref, w3r_ref, w2r_ref, out_ref):
    s = pl.program_id(0)
    xv = x_ref[...]
    iota = jax.lax.broadcasted_iota(jnp.int32, (_N, _L), 1)
    wtok = jnp.sum(jnp.where(iota == s, g_ref[...], 0.0),
                   axis=1, keepdims=True)
    h = jax.nn.silu(jnp.dot(xv, w1r_ref[0], preferred_element_type=jnp.float32)
                    * jnp.dot(xv, w3r_ref[0], preferred_element_type=jnp.float32))
    contrib = jnp.dot(wtok * h, w2r_ref[0], preferred_element_type=jnp.float32)

    @pl.when(s == 0)
    def _init():
        out_ref[...] = prev_ref[...] + contrib

    @pl.when(s > 0)
    def _acc():
        out_ref[...] += contrib


@functools.partial(jax.jit, static_argnames=())
def kernel(x, w1_shared, w2_shared, w3_shared, w1_routed, w2_routed, w3_routed,
           gate_w, gate_b):
    x2d = x.reshape(_N, _D)
    gwT_pad = jnp.pad(gate_w.T, ((0, 0), (0, _L - _E)))
    gb_pad = jnp.pad(gate_b, (0, _L - _E))

    logits16 = pl.pallas_call(
        _gate_body,
        out_shape=jax.ShapeDtypeStruct((_N, _L), jnp.float32),
    )(x2d, gwT_pad)

    shared2d = pl.pallas_call(
        _shared_body,
        grid=(_NS,),
        in_specs=[
            pl.BlockSpec((_N, _D), lambda i: (0, 0)),
            pl.BlockSpec((_D, _I), lambda i: (0, i)),
            pl.BlockSpec((_D, _I), lambda i: (0, i)),
            pl.BlockSpec((_I, _D), lambda i: (i, 0)),
        ],
        out_specs=pl.BlockSpec((_N, _D), lambda i: (0, 0)),
        out_shape=jax.ShapeDtypeStruct((_N, _D), jnp.float32),
        compiler_params=pltpu.CompilerParams(
            dimension_semantics=("arbitrary",),
        ),
    )(x2d, w1_shared, w3_shared, w2_shared)

    scores16, g16 = _routing(logits16, gb_pad)

    out2d = pl.pallas_call(
        _routed_body,
        grid=(_E,),
        in_specs=[
            pl.BlockSpec((_N, _D), lambda i: (0, 0)),
            pl.BlockSpec((_N, _L), lambda i: (0, 0)),
            pl.BlockSpec((_N, _D), lambda i: (0, 0)),
            pl.BlockSpec((1, _D, _I), lambda i: (i, 0, 0)),
            pl.BlockSpec((1, _D, _I), lambda i: (i, 0, 0)),
            pl.BlockSpec((1, _I, _D), lambda i: (i, 0, 0)),
        ],
        out_specs=pl.BlockSpec((_N, _D), lambda i: (0, 0)),
        out_shape=jax.ShapeDtypeStruct((_N, _D), jnp.float32),
        compiler_params=pltpu.CompilerParams(
            dimension_semantics=("arbitrary",),
        ),
    )(x2d, g16, shared2d, w1_routed, w3_routed, w2_routed)

    return (out2d.reshape(_B, _T, _D), scores16[:, :_E].reshape(_B, _T, _E))
